# baseline (speedup 1.0000x reference)
import functools

import jax
import jax.numpy as jnp
from jax import lax
from jax.experimental import pallas as pl
from jax.experimental.pallas import tpu as pltpu
from jax.experimental.pallas import tpu_sc as plsc

VOCAB_SIZE = 1000000
N_EMBD = 64
BATCH = 4096
SEQ_LEN = 200

NC, NS = 2, 16
NW = NC * NS
B_TOTAL = BATCH * SEQ_LEN
BPW = B_TOTAL // NW
CHUNK = 128
NCHUNK = BPW // CHUNK

_mesh = plsc.VectorSubcoreMesh(
    core_axis_name="c", subcore_axis_name="s", num_cores=NC, num_subcores=NS)


@functools.partial(
    pl.kernel,
    out_type=jax.ShapeDtypeStruct((B_TOTAL, N_EMBD), jnp.float32),
    mesh=_mesh,
    compiler_params=pltpu.CompilerParams(use_tc_tiling_on_sc=False),
    scratch_types=[
        pltpu.VMEM((NCHUNK, CHUNK), jnp.int32),
        pltpu.VMEM((CHUNK, N_EMBD), jnp.float32),
        pltpu.SemaphoreType.DMA,
    ],
)
def _embed_sc(idx_hbm, table_hbm, out_hbm, idx_v, rows_v, sem):
    wid = lax.axis_index("s") * NC + lax.axis_index("c")
    base = wid * BPW
    pltpu.sync_copy(idx_hbm.at[wid], idx_v)

    def body(i, carry):
        pltpu.async_copy(table_hbm.at[idx_v.at[i]], rows_v, sem).wait()
        pltpu.sync_copy(rows_v, out_hbm.at[pl.ds(base + i * CHUNK, CHUNK)])
        return carry

    lax.fori_loop(0, NCHUNK, body, 0)


def kernel(input_ids, weight):
    flat = input_ids.reshape(NW, NCHUNK, CHUNK)
    out = _embed_sc(flat, weight)
    return out.reshape(BATCH, SEQ_LEN, N_EMBD)


# speedup vs baseline: 1.1169x; 1.1169x over previous
import functools

import jax
import jax.numpy as jnp
from jax import lax
from jax.experimental import pallas as pl
from jax.experimental.pallas import tpu as pltpu
from jax.experimental.pallas import tpu_sc as plsc

VOCAB_SIZE = 1000000
N_EMBD = 64
BATCH = 4096
SEQ_LEN = 200

NC, NS = 2, 16
NW = NC * NS
B_TOTAL = BATCH * SEQ_LEN
BPW = B_TOTAL // NW
CHUNK = 128
NCHUNK = BPW // CHUNK
K = 2
BANKR = K * CHUNK
NGROUP = NCHUNK // K
NBANK = 4

_mesh = plsc.VectorSubcoreMesh(
    core_axis_name="c", subcore_axis_name="s", num_cores=NC, num_subcores=NS)


@functools.partial(
    pl.kernel,
    out_type=jax.ShapeDtypeStruct((B_TOTAL, N_EMBD), jnp.float32),
    mesh=_mesh,
    compiler_params=pltpu.CompilerParams(use_tc_tiling_on_sc=False),
    scratch_types=[
        pltpu.VMEM((NCHUNK, CHUNK), jnp.int32),
        pltpu.VMEM((NBANK, BANKR, N_EMBD), jnp.float32),
        pltpu.SemaphoreType.DMA,
        pltpu.SemaphoreType.DMA,
        pltpu.SemaphoreType.DMA,
        pltpu.SemaphoreType.DMA,
        pltpu.SemaphoreType.DMA,
        pltpu.SemaphoreType.DMA,
        pltpu.SemaphoreType.DMA,
        pltpu.SemaphoreType.DMA,
    ],
)
def _embed_sc(idx_hbm, table_hbm, out_hbm, idx_v, rows_v,
              g0, g1, g2, g3, s0, s1, s2, s3):
    gsems = (g0, g1, g2, g3)
    ssems = (s0, s1, s2, s3)
    wid = lax.axis_index("s") * NC + lax.axis_index("c")
    base = wid * BPW
    pltpu.sync_copy(idx_hbm.at[wid], idx_v)

    def fire_g(g, bank):
        for b in range(K):
            pltpu.async_copy(table_hbm.at[idx_v.at[g * K + b]],
                             rows_v.at[bank, pl.ds(b * CHUNK, CHUNK)],
                             gsems[bank])

    def wait_g(bank):
        pltpu.make_async_copy(table_hbm.at[pl.ds(0, BANKR)],
                              rows_v.at[bank], gsems[bank]).wait()

    def fire_s(g, bank):
        pltpu.async_copy(rows_v.at[bank],
                         out_hbm.at[pl.ds(base + g * BANKR, BANKR)],
                         ssems[bank])

    def wait_s(g, bank):
        pltpu.make_async_copy(rows_v.at[bank],
                              out_hbm.at[pl.ds(base + g * BANKR, BANKR)],
                              ssems[bank]).wait()

    fire_g(0, 0)
    fire_g(1, 1)
    wait_g(0); fire_s(0, 0); fire_g(2, 2)
    wait_g(1); fire_s(1, 1); fire_g(3, 3)

    def body(p, carry):
        for j in range(NBANK):
            g = NBANK * p + 2 + j
            bank = (2 + j) % NBANK
            nxt = j % NBANK
            wait_g(bank)
            fire_s(g, bank)
            wait_s(g - 2, nxt)
            fire_g(g + 2, nxt)
        return carry

    lax.fori_loop(0, (NGROUP - 4) // NBANK, body, 0)

    wait_g(2); fire_s(NGROUP - 2, 2); wait_s(NGROUP - 4, 0)
    wait_g(3); fire_s(NGROUP - 1, 3); wait_s(NGROUP - 3, 1)
    wait_s(NGROUP - 2, 2)
    wait_s(NGROUP - 1, 3)


def kernel(input_ids, weight):
    flat = input_ids.reshape(NW, NCHUNK, CHUNK)
    out = _embed_sc(flat, weight)
    return out.reshape(BATCH, SEQ_LEN, N_EMBD)
